# Initial kernel scaffold; baseline (speedup 1.0000x reference)
import jax
import jax.numpy as jnp
from jax.experimental import pallas as pl


def kernel(x, table):
    raise NotImplementedError("write your pallas kernel here")


# speedup vs baseline: 5.1171x; 5.1171x over previous
import functools

import numpy as np
import jax
import jax.numpy as jnp
from jax import lax
from jax.experimental import pallas as pl
from jax.experimental.pallas import tpu as pltpu
from jax.experimental.pallas import tpu_sc as plsc

_RES = 6
_GMIN = -2.0
_GMAX = 2.0
_STEP = (_GMAX - _GMIN) / (_RES - 1)

_G_HOST = np.zeros((512, 64), np.float32)
for _j in range(512):
    _G_HOST[_j, _j // 8] = float(_RES ** (7 - (_j % 8)))

_IDX_BLOCK = 1024


def _quant_body(x_ref, g_ref, o_ref):
    xc = jnp.clip(x_ref[...], _GMIN, _GMAX)
    t = jnp.round((xc - _GMIN) / _STEP)
    f = lax.dot_general(
        t, g_ref[...], (((1,), (0,)), ((), ())),
        precision=lax.Precision.HIGHEST,
        preferred_element_type=jnp.float32)
    o_ref[...] = f.astype(jnp.int32)


def _quantize(x2d, g):
    m, n = x2d.shape
    grid = m // _IDX_BLOCK
    return pl.pallas_call(
        _quant_body,
        grid=(grid,),
        in_specs=[
            pl.BlockSpec((_IDX_BLOCK, n), lambda i: (i, 0)),
            pl.BlockSpec((n, n // 8), lambda i: (0, 0)),
        ],
        out_specs=pl.BlockSpec((_IDX_BLOCK, n // 8), lambda i: (i, 0)),
        out_shape=jax.ShapeDtypeStruct((m, n // 8), jnp.int32),
        compiler_params=pltpu.CompilerParams(
            dimension_semantics=("parallel",)),
    )(x2d, g)


_NC = 2
_NS = 16
_NW = _NC * _NS
_CH = 16


def _gather_sc(table_f32, idx2d):
    nrows = idx2d.shape[0]
    rpw = nrows // _NW
    mesh = plsc.VectorSubcoreMesh(core_axis_name="c", subcore_axis_name="s")

    @functools.partial(
        pl.kernel,
        mesh=mesh,
        out_type=jax.ShapeDtypeStruct((nrows, 128, 4), jnp.float32),
        scratch_types=[
            pltpu.VMEM((_CH, 128), jnp.int32),
            pltpu.VMEM((_CH, 128, 4), jnp.float32),
            pltpu.SemaphoreType.DMA,
        ],
        compiler_params=pltpu.CompilerParams(use_tc_tiling_on_sc=False),
    )
    def gather_kernel(table_hbm, idx_hbm, out_hbm, idx_v, rows_v, sem):
        wid = lax.axis_index("s") * _NC + lax.axis_index("c")

        @pl.loop(0, rpw, step=_CH)
        def _chunk(r0):
            base = wid * rpw + r0
            pltpu.sync_copy(idx_hbm.at[pl.ds(base, _CH)], idx_v)
            copies = [
                pltpu.async_copy(table_hbm.at[idx_v.at[j]], rows_v.at[j], sem)
                for j in range(_CH)
            ]
            for cp in copies:
                cp.wait()
            pltpu.sync_copy(rows_v, out_hbm.at[pl.ds(base, _CH)])

    return gather_kernel(table_f32, idx2d)


def kernel(x, table):
    b, s, d = x.shape
    x2d = x.reshape(b, s * d)
    g = jnp.asarray(_G_HOST)
    idx = _quantize(x2d, g)
    idx2d = idx.reshape(-1, 128)
    tab32 = lax.bitcast_convert_type(
        table.reshape(-1, 4, 2), jnp.float32)
    out32 = _gather_sc(tab32, idx2d)
    y16 = lax.bitcast_convert_type(out32, jnp.float16)
    return y16.reshape(x.shape).astype(jnp.float32)


# speedup vs baseline: 11.6214x; 2.2711x over previous
import functools

import numpy as np
import jax
import jax.numpy as jnp
from jax import lax
from jax.experimental import pallas as pl
from jax.experimental.pallas import tpu as pltpu
from jax.experimental.pallas import tpu_sc as plsc

_RES = 6
_GMIN = -2.0
_GMAX = 2.0
_STEP = (_GMAX - _GMIN) / (_RES - 1)

_G_HOST = np.zeros((1024, 128), np.float32)
for _j in range(1024):
    _G_HOST[_j, _j // 8] = float(_RES ** (7 - (_j % 8)))

_IDX_BLOCK = 512


def _quant_body(x_ref, g_ref, o_ref):
    xc = jnp.clip(x_ref[...], _GMIN, _GMAX)
    t = jnp.round((xc - _GMIN) / _STEP)
    f = lax.dot_general(
        t, g_ref[...], (((1,), (0,)), ((), ())),
        precision=lax.Precision.HIGHEST,
        preferred_element_type=jnp.float32)
    o_ref[...] = f.astype(jnp.int32)


def _quantize(x2d, g):
    m, n = x2d.shape
    grid = m // _IDX_BLOCK
    return pl.pallas_call(
        _quant_body,
        grid=(grid,),
        in_specs=[
            pl.BlockSpec((_IDX_BLOCK, n), lambda i: (i, 0)),
            pl.BlockSpec((n, n // 8), lambda i: (0, 0)),
        ],
        out_specs=pl.BlockSpec((_IDX_BLOCK, n // 8), lambda i: (i, 0)),
        out_shape=jax.ShapeDtypeStruct((m, n // 8), jnp.int32),
        compiler_params=pltpu.CompilerParams(
            dimension_semantics=("parallel",)),
    )(x2d, g)


_NC = 2
_NS = 16
_NW = _NC * _NS
_CH = 16


def _gather_sc(table_f32, idx2d):
    nrows = idx2d.shape[0]
    rpw = nrows // _NW
    nout = nrows * 128
    mesh = plsc.VectorSubcoreMesh(core_axis_name="c", subcore_axis_name="s")

    @functools.partial(
        pl.kernel,
        mesh=mesh,
        out_type=jax.ShapeDtypeStruct((nout, 8), jnp.float32),
        scratch_types=[
            pltpu.VMEM((_CH, 128), jnp.int32),
            pltpu.VMEM((_CH * 128, 8), jnp.float32),
            pltpu.SemaphoreType.DMA,
        ],
        compiler_params=pltpu.CompilerParams(use_tc_tiling_on_sc=False),
    )
    def gather_kernel(table_hbm, idx_hbm, out_hbm, idx_v, rows_v, sem):
        wid = lax.axis_index("s") * _NC + lax.axis_index("c")

        @pl.loop(0, rpw, step=_CH)
        def _chunk(r0):
            base = wid * rpw + r0
            pltpu.sync_copy(idx_hbm.at[pl.ds(base, _CH)], idx_v)
            copies = [
                pltpu.async_copy(
                    table_hbm.at[idx_v.at[j]],
                    rows_v.at[pl.ds(j * 128, 128)], sem)
                for j in range(_CH)
            ]
            for cp in copies:
                cp.wait()
            pltpu.sync_copy(rows_v, out_hbm.at[pl.ds(base * 128, _CH * 128)])

    return gather_kernel(table_f32, idx2d)


def kernel(x, table):
    b, s, d = x.shape
    x2d = x.reshape(b // 2, 2 * s * d)
    g = jnp.asarray(_G_HOST)
    idx2d = _quantize(x2d, g)
    table_f32 = table.astype(jnp.float32)
    y = _gather_sc(table_f32, idx2d)
    return y.reshape(x.shape)
